# baseline (speedup 1.0000x reference)
import jax
import jax.numpy as jnp
from jax.experimental import pallas as pl
import jax.experimental.pallas.tpu as pltpu

N = 4096
NFEAT = 128
NHID = 32
NCLASS = 16
TILE = 512
NSPLIT = 2
SUB = TILE // NSPLIT


def _gcn_kernel(x_ref, *rest):
    adj_refs = rest[:NSPLIT]
    w1_ref, b1_ref, w2_ref, b2_ref, out_ref, s1_ref, s2_ref = rest[NSPLIT:]
    p = pl.program_id(0)
    t = pl.program_id(1)

    @pl.when((p == 0) & (t == 0))
    def _():
        s1 = jnp.dot(x_ref[...], w1_ref[...],
                     preferred_element_type=jnp.float32)
        s1_ref[...] = s1.astype(jnp.bfloat16)

    @pl.when(p == 0)
    def _():
        for i, aref in enumerate(adj_refs):
            a = aref[0].astype(jnp.bfloat16)
            h = jnp.dot(a, s1_ref[...], preferred_element_type=jnp.float32)
            h = jnp.maximum(h + b1_ref[...], 0.0).astype(jnp.bfloat16)
            s2_ref[pl.ds(t * TILE + i * SUB, SUB), :] = jnp.dot(
                h, w2_ref[...].astype(jnp.bfloat16),
                preferred_element_type=jnp.float32).astype(jnp.bfloat16)

    @pl.when(p == 1)
    def _():
        for i, aref in enumerate(adj_refs):
            a = aref[0].astype(jnp.bfloat16)
            o = jnp.dot(a, s2_ref[...], preferred_element_type=jnp.float32)
            o = o + b2_ref[...]
            out_ref[pl.ds(i * SUB, SUB), :] = (
                jnp.minimum(o, 0.0) - jnp.log1p(jnp.exp(-jnp.abs(o))))


@jax.jit
def kernel(x, adj_list, W1, b1, W2, b2):
    grid = (2, N // TILE)
    adj_specs = [
        pl.BlockSpec((1, SUB, N),
                     lambda p, t, i=i: (p, NSPLIT * t + i, 0))
        for i in range(NSPLIT)
    ]
    return pl.pallas_call(
        _gcn_kernel,
        grid=grid,
        in_specs=[
            pl.BlockSpec((N, NFEAT), lambda p, t: (0, 0)),
            *adj_specs,
            pl.BlockSpec((NFEAT, NHID), lambda p, t: (0, 0)),
            pl.BlockSpec((1, NHID), lambda p, t: (0, 0)),
            pl.BlockSpec((NHID, NCLASS), lambda p, t: (0, 0)),
            pl.BlockSpec((1, NCLASS), lambda p, t: (0, 0)),
        ],
        out_specs=pl.BlockSpec((TILE, NCLASS), lambda p, t: (t, 0)),
        out_shape=jax.ShapeDtypeStruct((N, NCLASS), jnp.float32),
        scratch_shapes=[
            pltpu.VMEM((N, NHID), jnp.bfloat16),
            pltpu.VMEM((N, NCLASS), jnp.bfloat16),
        ],
    )(x, *([adj_list] * NSPLIT), W1, b1.reshape(1, NHID), W2,
      b2.reshape(1, NCLASS))


# speedup vs baseline: 1.0440x; 1.0440x over previous
import jax
import jax.numpy as jnp
from jax.experimental import pallas as pl
import jax.experimental.pallas.tpu as pltpu

N = 4096
NFEAT = 128
NHID = 32
NCLASS = 16
TILE = 512


def _gcn_kernel(x_ref, adj_ref, w1_ref, b1_ref, w2_ref, b2_ref, out_ref,
                s1_ref, s2_ref):
    p = pl.program_id(0)
    t = pl.program_id(1)

    @pl.when((p == 0) & (t == 0))
    def _():
        s1 = jnp.dot(x_ref[...], w1_ref[...],
                     preferred_element_type=jnp.float32)
        s1_ref[...] = s1.astype(jnp.bfloat16)

    @pl.when(p == 0)
    def _():
        a = adj_ref[0].astype(jnp.bfloat16)
        h = jnp.dot(a, s1_ref[...], preferred_element_type=jnp.float32)
        h = jnp.maximum(h + b1_ref[...], 0.0).astype(jnp.bfloat16)
        s2_ref[pl.ds(t * TILE, TILE), :] = jnp.dot(
            h, w2_ref[...].astype(jnp.bfloat16),
            preferred_element_type=jnp.float32).astype(jnp.bfloat16)

    @pl.when(p == 1)
    def _():
        a = adj_ref[0].astype(jnp.bfloat16)
        o = jnp.dot(a, s2_ref[...], preferred_element_type=jnp.float32)
        o = o + b2_ref[...]
        out_ref[...] = jnp.minimum(o, 0.0) - jnp.log1p(jnp.exp(-jnp.abs(o)))


@jax.jit
def kernel(x, adj_list, W1, b1, W2, b2):
    grid = (2, N // TILE)
    return pl.pallas_call(
        _gcn_kernel,
        grid=grid,
        in_specs=[
            pl.BlockSpec((N, NFEAT), lambda p, t: (0, 0)),
            pl.BlockSpec((1, TILE, N), lambda p, t: (p, t, 0)),
            pl.BlockSpec((NFEAT, NHID), lambda p, t: (0, 0)),
            pl.BlockSpec((1, NHID), lambda p, t: (0, 0)),
            pl.BlockSpec((NHID, NCLASS), lambda p, t: (0, 0)),
            pl.BlockSpec((1, NCLASS), lambda p, t: (0, 0)),
        ],
        out_specs=pl.BlockSpec((TILE, NCLASS), lambda p, t: (p * t, 0)),
        out_shape=jax.ShapeDtypeStruct((N, NCLASS), jnp.float32),
        scratch_shapes=[
            pltpu.VMEM((N, NHID), jnp.bfloat16),
            pltpu.VMEM((N, NCLASS), jnp.bfloat16),
        ],
    )(x, adj_list, W1, b1.reshape(1, NHID), W2, b2.reshape(1, NCLASS))
